# Initial kernel scaffold; baseline (speedup 1.0000x reference)
import jax
import jax.numpy as jnp
from jax.experimental import pallas as pl


def kernel(x, edge_index, W_in, b_in, W_out, b_out, gpr_weights):
    raise NotImplementedError("write your pallas kernel here")


# speedup vs baseline: 4.3253x; 4.3253x over previous
import functools

import jax
import jax.numpy as jnp
from jax import lax
from jax.experimental import pallas as pl
from jax.experimental.pallas import tpu as pltpu
from jax.experimental.pallas import tpu_sc as plsc

N = 10000
E = 320000
H = 128
K = 10

NC = 2
NS = 16
NW = NC * NS
CH = 128
EPT = 10240
NCH = EPT // CH
EPAD = NW * EPT
NP = 10240
RPS = NP // NS
ZR = CH

_mesh = plsc.VectorSubcoreMesh(core_axis_name="c", subcore_axis_name="s")


def _edge_body(u_hbm, idx_hbm, out_hbm,
               ib0, ib1, g0, g1, acc, si0, si1, sg0, sg1, ss0, ss1):
    c = lax.axis_index("c")
    s = lax.axis_index("s")
    w = c * NS + s

    zero = jnp.zeros((16,), jnp.float32)

    @pl.loop(0, ZR)
    def _(i):
        @pl.loop(0, H // 16)
        def _(l):
            g0[i, pl.ds(l * 16, 16)] = zero

    @pl.loop(0, RPS // ZR)
    def _(k):
        pltpu.sync_copy(g0, acc.at[pl.ds(s * RPS + k * ZR, ZR)])

    plsc.subcore_barrier()

    def idx(j, ib, sem):
        pltpu.async_copy(idx_hbm.at[w, j], ib, sem)

    def idx_wait(j, ib, sem):
        pltpu.make_async_copy(idx_hbm.at[w, j], ib, sem).wait()

    def gat(ib, buf, sem):
        pltpu.async_copy(u_hbm.at[ib.at[0]], buf, sem)

    def gat_wait(ib, buf, sem):
        pltpu.make_async_copy(u_hbm.at[ib.at[0]], buf, sem).wait()

    def sca(ib, buf, sem):
        pltpu.async_copy(buf, acc.at[ib.at[1]], sem, add=True)

    def sca_wait(ib, buf, sem):
        pltpu.make_async_copy(buf, acc.at[ib.at[1]], sem).wait()

    idx(0, ib0, si0)
    idx(1, ib1, si1)
    idx_wait(0, ib0, si0)
    gat(ib0, g0, sg0)
    idx_wait(1, ib1, si1)
    gat(ib1, g1, sg1)

    @pl.loop(0, NCH, step=2)
    def _(j):
        gat_wait(ib0, g0, sg0)
        sca(ib0, g0, ss0)
        gat_wait(ib1, g1, sg1)
        sca(ib1, g1, ss1)
        sca_wait(ib0, g0, ss0)

        @pl.when(j + 2 < NCH)
        def _():
            idx(j + 2, ib0, si0)

        sca_wait(ib1, g1, ss1)

        @pl.when(j + 3 < NCH)
        def _():
            idx(j + 3, ib1, si1)

        @pl.when(j + 2 < NCH)
        def _():
            idx_wait(j + 2, ib0, si0)
            gat(ib0, g0, sg0)

        @pl.when(j + 3 < NCH)
        def _():
            idx_wait(j + 3, ib1, si1)
            gat(ib1, g1, sg1)

    plsc.subcore_barrier()
    pltpu.sync_copy(acc.at[pl.ds(s * RPS, RPS)],
                    out_hbm.at[c, pl.ds(s * RPS, RPS)])


_edge_call = functools.partial(
    pl.kernel,
    out_type=jax.ShapeDtypeStruct((NC, NP, H), jnp.float32),
    mesh=_mesh,
    scratch_types=[
        pltpu.VMEM((2, CH), jnp.int32),
        pltpu.VMEM((2, CH), jnp.int32),
        pltpu.VMEM((CH, H), jnp.float32),
        pltpu.VMEM((CH, H), jnp.float32),
        pltpu.VMEM_SHARED((NP, H), jnp.float32),
        pltpu.SemaphoreType.DMA,
        pltpu.SemaphoreType.DMA,
        pltpu.SemaphoreType.DMA,
        pltpu.SemaphoreType.DMA,
        pltpu.SemaphoreType.DMA,
        pltpu.SemaphoreType.DMA,
    ],
)(_edge_body)


_R = 2000


def _init_body(x_ref, wt_ref, b_ref, d0_ref, d1_ref, w0_ref,
               u_ref, s_ref, dinv_ref, sq_ref):
    h0 = jnp.dot(x_ref[...], wt_ref[...], preferred_element_type=jnp.float32)
    h0 = jnp.maximum(h0 + b_ref[...], 0.0)
    deg = d0_ref[0] + d1_ref[0] + 1.0
    dis = lax.rsqrt(deg)
    u0 = h0 * dis
    u_ref[...] = u0
    s_ref[...] = w0_ref[0, 0] * u0
    dinv_ref[...] = 1.0 / deg
    sq_ref[...] = deg * dis


def _init_call(x, wt, b2, degacc, w0):
    fs = jax.ShapeDtypeStruct((N, H), jnp.float32)
    return pl.pallas_call(
        _init_body,
        grid=(N // _R,),
        in_specs=[
            pl.BlockSpec((_R, H), lambda i: (i, 0)),
            pl.BlockSpec((H, H), lambda i: (0, 0)),
            pl.BlockSpec((1, H), lambda i: (0, 0)),
            pl.BlockSpec((1, _R, H), lambda i: (0, i, 0)),
            pl.BlockSpec((1, _R, H), lambda i: (1, i, 0)),
            pl.BlockSpec(memory_space=pltpu.SMEM),
        ],
        out_specs=[pl.BlockSpec((_R, H), lambda i: (i, 0))] * 4,
        out_shape=[fs, fs, fs, fs],
    )(x, wt, b2, degacc, degacc, w0)


def _combine_body(a_ref, u_ref, s_ref, dinv_ref, wk_ref, uo_ref, so_ref):
    t = (a_ref[0] + a_ref[1] + u_ref[...]) * dinv_ref[...]
    uo_ref[...] = t
    so_ref[...] = s_ref[...] + wk_ref[0, 0] * t


def _combine_call(accs, u, s, dinv, wk):
    fs = jax.ShapeDtypeStruct((N, H), jnp.float32)
    return pl.pallas_call(
        _combine_body,
        grid=(N // _R,),
        in_specs=[
            pl.BlockSpec((NC, _R, H), lambda i: (0, i, 0)),
            pl.BlockSpec((_R, H), lambda i: (i, 0)),
            pl.BlockSpec((_R, H), lambda i: (i, 0)),
            pl.BlockSpec((_R, H), lambda i: (i, 0)),
            pl.BlockSpec(memory_space=pltpu.SMEM),
        ],
        out_specs=[pl.BlockSpec((_R, H), lambda i: (i, 0))] * 2,
        out_shape=[fs, fs],
    )(accs, u, s, dinv, wk)


def _final_body(s_ref, sq_ref, wt_ref, b_ref, o_ref):
    hf = s_ref[...] * sq_ref[...]
    o_ref[...] = jnp.dot(hf, wt_ref[...],
                         preferred_element_type=jnp.float32) + b_ref[...]


def _final_call(s, sq, wt_pad, b_pad):
    return pl.pallas_call(
        _final_body,
        grid=(N // _R,),
        in_specs=[
            pl.BlockSpec((_R, H), lambda i: (i, 0)),
            pl.BlockSpec((_R, H), lambda i: (i, 0)),
            pl.BlockSpec((H, H), lambda i: (0, 0)),
            pl.BlockSpec((1, H), lambda i: (0, 0)),
        ],
        out_specs=pl.BlockSpec((_R, H), lambda i: (i, 0)),
        out_shape=jax.ShapeDtypeStruct((N, H), jnp.float32),
    )(s, sq, wt_pad, b_pad)


def kernel(x, edge_index, W_in, b_in, W_out, b_out, gpr_weights):
    row = edge_index[0]
    col = edge_index[1]
    pad = EPAD - E
    row3 = jnp.concatenate(
        [row, jnp.full((pad,), N, jnp.int32)]).reshape(NW, NCH, CH)
    col3 = jnp.concatenate(
        [col, jnp.zeros((pad,), jnp.int32)]).reshape(NW, NCH, CH)
    idx3 = jnp.stack([col3, row3], axis=2)

    sw = jax.nn.softmax(gpr_weights.astype(jnp.float32))
    wt_in = W_in.T
    wt_out = jnp.zeros((H, H), jnp.float32).at[:, :2].set(W_out.T)
    b_out_pad = jnp.zeros((1, H), jnp.float32).at[0, :2].set(b_out)
    b_in2 = b_in.reshape(1, H)

    ones = jnp.ones((N, H), jnp.float32)
    degacc = _edge_call(ones, idx3)

    u, s, dinv, sq = _init_call(x, wt_in, b_in2, degacc,
                                sw[0].reshape(1, 1))

    for k in range(1, K + 1):
        accs = _edge_call(u, idx3)
        u, s = _combine_call(accs, u, s, dinv, sw[k].reshape(1, 1))

    out = _final_call(s, sq, wt_out, b_out_pad)
    return out[:, :2]
